# baseline (speedup 1.0000x reference)
import functools

import jax
import jax.numpy as jnp
from jax.experimental import pallas as pl
from jax.experimental.pallas import tpu as pltpu

N_X = 4096
N_T = 4096
ROWS = 512
CHUNK = 512


def _meshgrid_kernel(x_col, t_row, t_col, xx_ref, tt_ref, ts_s, rank_s):
    i = pl.program_id(0)

    @pl.when(i == 0)
    def _sort():
        tr = t_row[:]
        j_idx = jax.lax.broadcasted_iota(jnp.int32, (1, N_T), 1)
        for k in range(N_T // CHUNK):
            ti = t_col[pl.ds(k * CHUNK, CHUNK), :]
            i_idx = (k * CHUNK
                     + jax.lax.broadcasted_iota(jnp.int32, (CHUNK, 1), 0))
            less = (tr < ti) | ((tr == ti) & (j_idx < i_idx))
            rank_s[pl.ds(k * CHUNK, CHUNK), :] = jnp.sum(
                less.astype(jnp.int32), axis=1, keepdims=True)
        tc = t_col[:]
        rk = rank_s[:]
        for k in range(N_T // CHUNK):
            r_idx = (k * CHUNK
                     + jax.lax.broadcasted_iota(jnp.int32, (1, CHUNK), 1))
            sel = jnp.where(rk == r_idx, tc, 0.0)
            ts_s[0, pl.ds(k * CHUNK, CHUNK)] = jnp.sum(sel, axis=0)

    xx_ref[:] = jnp.broadcast_to(x_col[:], (ROWS, N_T))
    tt_ref[:] = jnp.broadcast_to(ts_s[:], (ROWS, N_T))


@jax.jit
def kernel(x_grid, t_grid):
    x_col = x_grid.reshape(N_X, 1)
    t_row = t_grid.reshape(1, N_T)
    t_col = t_grid.reshape(N_T, 1)
    grid = (N_X // ROWS,)
    xx, tt = pl.pallas_call(
        _meshgrid_kernel,
        grid=grid,
        in_specs=[
            pl.BlockSpec((ROWS, 1), lambda i: (i, 0)),
            pl.BlockSpec((1, N_T), lambda i: (0, 0)),
            pl.BlockSpec((N_T, 1), lambda i: (0, 0)),
        ],
        out_specs=[
            pl.BlockSpec((ROWS, N_T), lambda i: (i, 0)),
            pl.BlockSpec((ROWS, N_T), lambda i: (i, 0)),
        ],
        out_shape=[
            jax.ShapeDtypeStruct((N_X, N_T), jnp.float32),
            jax.ShapeDtypeStruct((N_X, N_T), jnp.float32),
        ],
        scratch_shapes=[
            pltpu.VMEM((1, N_T), jnp.float32),
            pltpu.VMEM((N_T, 1), jnp.int32),
        ],
    )(x_col, t_row, t_col)
    return (xx.reshape(-1, 1), tt.reshape(-1, 1))
